# Initial kernel scaffold; baseline (speedup 1.0000x reference)
import jax
import jax.numpy as jnp
from jax.experimental import pallas as pl


def kernel(x, beta, alpha, G, G_inv, eps):
    raise NotImplementedError("write your pallas kernel here")


# speedup vs baseline: 8.8008x; 8.8008x over previous
import jax
import jax.numpy as jnp
from jax.experimental import pallas as pl
from jax.experimental.pallas import tpu as pltpu

_Q = 4.0
_M = 6
_TINY = float(jnp.finfo(jnp.float32).eps)


def _cround(x):
    return jnp.floor(x - jnp.sign(x) * _TINY + 0.5)


def _g_x(xs, fs):
    best = jnp.abs(xs[0] - fs[0])
    k = jnp.zeros_like(best)
    xk = xs[0]
    fk = fs[0]
    for i in range(1, 8):
        d = jnp.abs(xs[i] - fs[i])
        c = d > best
        best = jnp.where(c, d, best)
        k = jnp.where(c, float(i), k)
        xk = jnp.where(c, xs[i], xk)
        fk = jnp.where(c, fs[i], fk)
    pos = xk >= 0.0
    cond = (pos & (fk < xk)) | (jnp.logical_not(pos) & (fk <= xk))
    nfk = fk + jnp.where(cond, 1.0, -1.0)
    return [jnp.where(k == float(i), nfk, f) for i, f in enumerate(fs)]


def _cpe8(xs):
    fs = [_cround(x) for x in xs]
    s0 = fs[0]
    for i in range(1, 8):
        s0 = s0 + fs[i]
    even0 = jnp.mod(s0, 2.0) == 0.0
    g0 = _g_x(xs, fs)
    y0 = [jnp.where(even0, f, g) for f, g in zip(fs, g0)]

    xs2 = [x - 0.5 for x in xs]
    fs2 = [_cround(x) for x in xs2]
    s1 = fs2[0]
    for i in range(1, 8):
        s1 = s1 + fs2[i]
    even1 = jnp.mod(s1, 2.0) == 0.0
    g1 = _g_x(xs2, fs2)
    y1 = [jnp.where(even1, f, g) + 0.5 for f, g in zip(fs2, g1)]

    d0 = (xs[0] - y0[0]) * (xs[0] - y0[0])
    d1 = (xs[0] - y1[0]) * (xs[0] - y1[0])
    for i in range(1, 8):
        d0 = d0 + (xs[i] - y0[i]) * (xs[i] - y0[i])
        d1 = d1 + (xs[i] - y1[i]) * (xs[i] - y1[i])
    c = d0 < d1
    return [jnp.where(c, a, b) for a, b in zip(y0, y1)]


def _encode_coords(xl):
    suf = [None] * 7
    suf[6] = xl[6]
    for j in range(5, 0, -1):
        suf[j] = xl[j] + suf[j + 1]
    c = [None] * 8
    c[0] = 0.5 * (xl[0] + suf[1]) - 3.5 * xl[7]
    for j in range(1, 7):
        c[j] = suf[j] - float(7 - j) * xl[7]
    c[7] = 2.0 * xl[7]
    return c


def _decode_Gb(b):
    h = 0.5 * b[7]
    Gb = [None] * 8
    Gb[0] = 2.0 * b[0] - b[1] + h
    for i in range(1, 6):
        Gb[i] = b[i] - b[i + 1] + h
    Gb[6] = b[6] + h
    Gb[7] = h
    return Gb


def _hnlq_body(beta_ref, eps_ref, x_ref, o_ref):
    beta = beta_ref[0]
    xs = [x_ref[i] for i in range(8)]
    t = [xs[i] / beta + eps_ref[i] for i in range(8)]
    xhat = None
    for layer in range(_M):
        xl = _cpe8(t)
        cc = _encode_coords(xl)
        b = [jnp.mod(_cround(v), _Q) for v in cc]
        t = [v * 0.25 for v in xl]
        Gb = _decode_Gb(b)
        gq = _cpe8([v * 0.25 for v in Gb])
        xi = [g - _Q * q for g, q in zip(Gb, gq)]
        if layer == 0:
            xhat = xi
        else:
            w = float(_Q ** layer)
            xhat = [h + w * v for h, v in zip(xhat, xi)]
    for i in range(8):
        o_ref[i] = beta * xhat[i]


def _hnlq_transposed(xt, beta, eps, sb):
    s = xt.shape[1]
    grid = s // sb
    return pl.pallas_call(
        _hnlq_body,
        grid=(grid,),
        in_specs=[
            pl.BlockSpec(memory_space=pltpu.SMEM),
            pl.BlockSpec(memory_space=pltpu.SMEM),
            pl.BlockSpec((8, sb, 128), lambda i: (0, i, 0)),
        ],
        out_specs=pl.BlockSpec((8, sb, 128), lambda i: (0, i, 0)),
        out_shape=jax.ShapeDtypeStruct(xt.shape, jnp.float32),
    )(beta, eps, xt)


def kernel(x, beta, alpha, G, G_inv, eps):
    n = x.shape[0]
    sb = 8
    assert n % (sb * 128) == 0
    xt = x.T.reshape(8, n // 128, 128)
    out = _hnlq_transposed(xt, jnp.reshape(beta, (1,)), eps, sb)
    x_hat = out.reshape(8, n).T
    return x + jax.lax.stop_gradient(x_hat - x)


# speedup vs baseline: 10.6451x; 1.2096x over previous
import jax
import jax.numpy as jnp
from jax.experimental import pallas as pl
from jax.experimental.pallas import tpu as pltpu

_Q = 4.0
_M = 6
_TINY = float(jnp.finfo(jnp.float32).eps)


def _cround(x):
    xb = jax.lax.bitcast_convert_type(x, jnp.uint32)
    st = (xb & jnp.uint32(0x80000000)) | jnp.uint32(0x34000000)
    y = x - jax.lax.bitcast_convert_type(st, jnp.float32)
    return jnp.floor(y + 0.5)


def _is_even(s):
    return (s.astype(jnp.int32) & 1) == 0


def _digit_mod4(v):
    return (v.astype(jnp.int32) & 3).astype(jnp.float32)


def _g_x(xs, fs):
    best = jnp.abs(xs[0] - fs[0])
    k = jnp.zeros_like(best)
    xk = xs[0]
    fk = fs[0]
    for i in range(1, 8):
        d = jnp.abs(xs[i] - fs[i])
        c = d > best
        best = jnp.where(c, d, best)
        k = jnp.where(c, float(i), k)
        xk = jnp.where(c, xs[i], xk)
        fk = jnp.where(c, fs[i], fk)
    pos = xk >= 0.0
    cond = (pos & (fk < xk)) | (jnp.logical_not(pos) & (fk <= xk))
    nfk = fk + jnp.where(cond, 1.0, -1.0)
    return [jnp.where(k == float(i), nfk, f) for i, f in enumerate(fs)]


def _cpe8(xs):
    fs = [_cround(x) for x in xs]
    s0 = fs[0]
    for i in range(1, 8):
        s0 = s0 + fs[i]
    even0 = _is_even(s0)
    g0 = _g_x(xs, fs)
    y0 = [jnp.where(even0, f, g) for f, g in zip(fs, g0)]

    xs2 = [x - 0.5 for x in xs]
    fs2 = [_cround(x) for x in xs2]
    s1 = fs2[0]
    for i in range(1, 8):
        s1 = s1 + fs2[i]
    even1 = _is_even(s1)
    g1 = _g_x(xs2, fs2)
    y1 = [jnp.where(even1, f, g) + 0.5 for f, g in zip(fs2, g1)]

    d0 = (xs[0] - y0[0]) * (xs[0] - y0[0])
    d1 = (xs[0] - y1[0]) * (xs[0] - y1[0])
    for i in range(1, 8):
        d0 = d0 + (xs[i] - y0[i]) * (xs[i] - y0[i])
        d1 = d1 + (xs[i] - y1[i]) * (xs[i] - y1[i])
    c = d0 < d1
    return [jnp.where(c, a, b) for a, b in zip(y0, y1)]


def _encode_coords(xl):
    suf = [None] * 7
    suf[6] = xl[6]
    for j in range(5, 0, -1):
        suf[j] = xl[j] + suf[j + 1]
    c = [None] * 8
    c[0] = 0.5 * (xl[0] + suf[1]) - 3.5 * xl[7]
    for j in range(1, 7):
        c[j] = suf[j] - float(7 - j) * xl[7]
    c[7] = 2.0 * xl[7]
    return c


def _decode_Gb(b):
    h = 0.5 * b[7]
    Gb = [None] * 8
    Gb[0] = 2.0 * b[0] - b[1] + h
    for i in range(1, 6):
        Gb[i] = b[i] - b[i + 1] + h
    Gb[6] = b[6] + h
    Gb[7] = h
    return Gb


def _hnlq_body(beta_ref, eps_ref, x_ref, o_ref):
    beta = beta_ref[0]
    xs = [x_ref[i] for i in range(8)]
    t = [xs[i] / beta + eps_ref[i] for i in range(8)]
    xhat = None
    for layer in range(_M):
        xl = _cpe8(t)
        cc = _encode_coords(xl)
        b = [_digit_mod4(v) for v in cc]
        t = [v * 0.25 for v in xl]
        Gb = _decode_Gb(b)
        gq = _cpe8([v * 0.25 for v in Gb])
        xi = [g - _Q * q for g, q in zip(Gb, gq)]
        if layer == 0:
            xhat = xi
        else:
            w = float(_Q ** layer)
            xhat = [h + w * v for h, v in zip(xhat, xi)]
    for i in range(8):
        o_ref[i] = beta * xhat[i]


def _hnlq_transposed(xt, beta, eps, sb):
    s = xt.shape[1]
    grid = s // sb
    return pl.pallas_call(
        _hnlq_body,
        grid=(grid,),
        in_specs=[
            pl.BlockSpec(memory_space=pltpu.SMEM),
            pl.BlockSpec(memory_space=pltpu.SMEM),
            pl.BlockSpec((8, sb, 128), lambda i: (0, i, 0)),
        ],
        out_specs=pl.BlockSpec((8, sb, 128), lambda i: (0, i, 0)),
        out_shape=jax.ShapeDtypeStruct(xt.shape, jnp.float32),
    )(beta, eps, xt)


def kernel(x, beta, alpha, G, G_inv, eps):
    n = x.shape[0]
    sb = 8
    assert n % (sb * 128) == 0
    xt = x.T.reshape(8, n // 128, 128)
    out = _hnlq_transposed(xt, jnp.reshape(beta, (1,)), eps, sb)
    x_hat = out.reshape(8, n).T
    return x + jax.lax.stop_gradient(x_hat - x)


# speedup vs baseline: 10.7482x; 1.0097x over previous
import jax
import jax.numpy as jnp
from jax.experimental import pallas as pl
from jax.experimental.pallas import tpu as pltpu

_Q = 4.0
_M = 6
_TINY = float(jnp.finfo(jnp.float32).eps)


def _cround(x):
    xb = jax.lax.bitcast_convert_type(x, jnp.uint32)
    st = (xb & jnp.uint32(0x80000000)) | jnp.uint32(0x34000000)
    y = x - jax.lax.bitcast_convert_type(st, jnp.float32)
    return jnp.floor(y + 0.5)


def _is_even(s):
    return (s.astype(jnp.int32) & 1) == 0


def _digit_mod4(v):
    return (v.astype(jnp.int32) & 3).astype(jnp.float32)


def _g_x_parts(xs, fs):
    s = xs[0] - fs[0]
    best = jnp.abs(s)
    k = jnp.zeros_like(best)
    sk = s
    fk = fs[0]
    for i in range(1, 8):
        si = xs[i] - fs[i]
        d = jnp.abs(si)
        c = d > best
        best = jnp.where(c, d, best)
        k = jnp.where(c, float(i), k)
        sk = jnp.where(c, si, sk)
        fk = jnp.where(c, fs[i], fk)
    cond = (sk > 0.0) | ((sk == 0.0) & (fk < 0.0))
    nfk = fk + jnp.where(cond, 1.0, -1.0)
    return k, nfk


def _cpe8(xs):
    fs = [_cround(x) for x in xs]
    s0 = fs[0]
    for i in range(1, 8):
        s0 = s0 + fs[i]
    even0 = _is_even(s0)
    k0, nf0 = _g_x_parts(xs, fs)
    k0 = jnp.where(even0, -1.0, k0)
    y0 = [jnp.where(k0 == float(i), nf0, f) for i, f in enumerate(fs)]

    xs2 = [x - 0.5 for x in xs]
    fs2 = [_cround(x) for x in xs2]
    s1 = fs2[0]
    for i in range(1, 8):
        s1 = s1 + fs2[i]
    even1 = _is_even(s1)
    k1, nf1 = _g_x_parts(xs2, fs2)
    k1 = jnp.where(even1, -1.0, k1)
    y1 = [jnp.where(k1 == float(i), nf1, f) + 0.5 for i, f in enumerate(fs2)]

    d0 = (xs[0] - y0[0]) * (xs[0] - y0[0])
    d1 = (xs[0] - y1[0]) * (xs[0] - y1[0])
    for i in range(1, 8):
        d0 = d0 + (xs[i] - y0[i]) * (xs[i] - y0[i])
        d1 = d1 + (xs[i] - y1[i]) * (xs[i] - y1[i])
    c = d0 < d1
    return [jnp.where(c, a, b) for a, b in zip(y0, y1)]


def _encode_coords(xl):
    suf = [None] * 7
    suf[6] = xl[6]
    for j in range(5, 0, -1):
        suf[j] = xl[j] + suf[j + 1]
    c = [None] * 8
    c[0] = 0.5 * (xl[0] + suf[1]) - 3.5 * xl[7]
    for j in range(1, 7):
        c[j] = suf[j] - float(7 - j) * xl[7]
    c[7] = 2.0 * xl[7]
    return c


def _decode_Gb(b):
    h = 0.5 * b[7]
    Gb = [None] * 8
    Gb[0] = 2.0 * b[0] - b[1] + h
    for i in range(1, 6):
        Gb[i] = b[i] - b[i + 1] + h
    Gb[6] = b[6] + h
    Gb[7] = h
    return Gb


def _hnlq_body(beta_ref, eps_ref, x_ref, o_ref):
    beta = beta_ref[0]
    xs = [x_ref[i] for i in range(8)]
    t = [xs[i] / beta + eps_ref[i] for i in range(8)]
    xhat = None
    for layer in range(_M):
        xl = _cpe8(t)
        cc = _encode_coords(xl)
        b = [_digit_mod4(v) for v in cc]
        t = [v * 0.25 for v in xl]
        Gb = _decode_Gb(b)
        gq = _cpe8([v * 0.25 for v in Gb])
        xi = [g - _Q * q for g, q in zip(Gb, gq)]
        if layer == 0:
            xhat = xi
        else:
            w = float(_Q ** layer)
            xhat = [h + w * v for h, v in zip(xhat, xi)]
    for i in range(8):
        o_ref[i] = beta * xhat[i]


def _hnlq_transposed(xt, beta, eps, sb):
    s = xt.shape[1]
    grid = s // sb
    return pl.pallas_call(
        _hnlq_body,
        grid=(grid,),
        in_specs=[
            pl.BlockSpec(memory_space=pltpu.SMEM),
            pl.BlockSpec(memory_space=pltpu.SMEM),
            pl.BlockSpec((8, sb, 128), lambda i: (0, i, 0)),
        ],
        out_specs=pl.BlockSpec((8, sb, 128), lambda i: (0, i, 0)),
        out_shape=jax.ShapeDtypeStruct(xt.shape, jnp.float32),
    )(beta, eps, xt)


def kernel(x, beta, alpha, G, G_inv, eps):
    n = x.shape[0]
    sb = 8
    assert n % (sb * 128) == 0
    xt = x.T.reshape(8, n // 128, 128)
    out = _hnlq_transposed(xt, jnp.reshape(beta, (1,)), eps, sb)
    x_hat = out.reshape(8, n).T
    return x + jax.lax.stop_gradient(x_hat - x)


# speedup vs baseline: 12.3752x; 1.1514x over previous
import jax
import jax.numpy as jnp
from jax.experimental import pallas as pl
from jax.experimental.pallas import tpu as pltpu

_Q = 4.0
_M = 6
_TINY = float(jnp.finfo(jnp.float32).eps)


def _cround(x):
    xb = jax.lax.bitcast_convert_type(x, jnp.uint32)
    st = (xb & jnp.uint32(0x80000000)) | jnp.uint32(0x34000000)
    y = x - jax.lax.bitcast_convert_type(st, jnp.float32)
    return jnp.floor(y + 0.5)


def _is_even(s):
    return (s.astype(jnp.int32) & 1) == 0


def _digit_mod4(v):
    return (v.astype(jnp.int32) & 3).astype(jnp.float32)


def _g_x_parts(xs, fs):
    s = xs[0] - fs[0]
    best = jnp.abs(s)
    k = jnp.zeros_like(best)
    sk = s
    fk = fs[0]
    for i in range(1, 8):
        si = xs[i] - fs[i]
        d = jnp.abs(si)
        c = d > best
        best = jnp.where(c, d, best)
        k = jnp.where(c, float(i), k)
        sk = jnp.where(c, si, sk)
        fk = jnp.where(c, fs[i], fk)
    cond = (sk > 0.0) | ((sk == 0.0) & (fk < 0.0))
    nfk = fk + jnp.where(cond, 1.0, -1.0)
    return k, nfk


def _cpe8(xs):
    fs = [_cround(x) for x in xs]
    s0 = fs[0]
    for i in range(1, 8):
        s0 = s0 + fs[i]
    even0 = _is_even(s0)
    k0, nf0 = _g_x_parts(xs, fs)
    k0 = jnp.where(even0, -1.0, k0)
    y0 = [jnp.where(k0 == float(i), nf0, f) for i, f in enumerate(fs)]

    xs2 = [x - 0.5 for x in xs]
    fs2 = [_cround(x) for x in xs2]
    s1 = fs2[0]
    for i in range(1, 8):
        s1 = s1 + fs2[i]
    even1 = _is_even(s1)
    k1, nf1 = _g_x_parts(xs2, fs2)
    k1 = jnp.where(even1, -1.0, k1)
    y1 = [jnp.where(k1 == float(i), nf1, f) + 0.5 for i, f in enumerate(fs2)]

    d0 = (xs[0] - y0[0]) * (xs[0] - y0[0])
    d1 = (xs[0] - y1[0]) * (xs[0] - y1[0])
    for i in range(1, 8):
        d0 = d0 + (xs[i] - y0[i]) * (xs[i] - y0[i])
        d1 = d1 + (xs[i] - y1[i]) * (xs[i] - y1[i])
    c = d0 < d1
    return [jnp.where(c, a, b) for a, b in zip(y0, y1)]


def _encode_coords(xl):
    suf = [None] * 7
    suf[6] = xl[6]
    for j in range(5, 0, -1):
        suf[j] = xl[j] + suf[j + 1]
    c = [None] * 8
    c[0] = 0.5 * (xl[0] + suf[1]) - 3.5 * xl[7]
    for j in range(1, 7):
        c[j] = suf[j] - float(7 - j) * xl[7]
    c[7] = 2.0 * xl[7]
    return c


def _decode_Gb(b):
    h = 0.5 * b[7]
    Gb = [None] * 8
    Gb[0] = 2.0 * b[0] - b[1] + h
    for i in range(1, 6):
        Gb[i] = b[i] - b[i + 1] + h
    Gb[6] = b[6] + h
    Gb[7] = h
    return Gb


def _hnlq_body(beta_ref, eps_ref, x_ref, o_ref):
    beta = beta_ref[0]
    xs = [x_ref[i] for i in range(8)]
    t = [xs[i] / beta + eps_ref[i] for i in range(8)]
    xhat = None
    for layer in range(_M):
        xl = _cpe8(t)
        cc = _encode_coords(xl)
        b = [_digit_mod4(v) for v in cc]
        t = [v * 0.25 for v in xl]
        Gb = _decode_Gb(b)
        gq = _cpe8([v * 0.25 for v in Gb])
        xi = [g - _Q * q for g, q in zip(Gb, gq)]
        if layer == 0:
            xhat = xi
        else:
            w = float(_Q ** layer)
            xhat = [h + w * v for h, v in zip(xhat, xi)]
    for i in range(8):
        o_ref[i] = beta * xhat[i]


def _hnlq_transposed(xt, beta, eps, sb):
    s = xt.shape[1]
    grid = s // sb
    return pl.pallas_call(
        _hnlq_body,
        grid=(grid,),
        in_specs=[
            pl.BlockSpec(memory_space=pltpu.SMEM),
            pl.BlockSpec(memory_space=pltpu.SMEM),
            pl.BlockSpec((8, sb, 128), lambda i: (0, i, 0)),
        ],
        out_specs=pl.BlockSpec((8, sb, 128), lambda i: (0, i, 0)),
        out_shape=jax.ShapeDtypeStruct(xt.shape, jnp.float32),
    )(beta, eps, xt)


def kernel(x, beta, alpha, G, G_inv, eps):
    n = x.shape[0]
    sb = 16
    assert n % (sb * 128) == 0
    xt = x.T.reshape(8, n // 128, 128)
    out = _hnlq_transposed(xt, jnp.reshape(beta, (1,)), eps, sb)
    x_hat = out.reshape(8, n).T
    return x + jax.lax.stop_gradient(x_hat - x)


# speedup vs baseline: 12.6044x; 1.0185x over previous
import jax
import jax.numpy as jnp
from jax.experimental import pallas as pl
from jax.experimental.pallas import tpu as pltpu

_Q = 4.0
_M = 6
_TINY = float(jnp.finfo(jnp.float32).eps)


def _cround(x):
    xb = jax.lax.bitcast_convert_type(x, jnp.uint32)
    st = (xb & jnp.uint32(0x80000000)) | jnp.uint32(0x34000000)
    y = x - jax.lax.bitcast_convert_type(st, jnp.float32)
    return jnp.floor(y + 0.5)


def _is_even(s):
    return (s.astype(jnp.int32) & 1) == 0


def _digit_mod4(v):
    return (v.astype(jnp.int32) & 3).astype(jnp.float32)


def _g_x_parts(xs, fs):
    s = xs[0] - fs[0]
    best = jnp.abs(s)
    k = jnp.zeros_like(best)
    sk = s
    fk = fs[0]
    for i in range(1, 8):
        si = xs[i] - fs[i]
        d = jnp.abs(si)
        c = d > best
        best = jnp.where(c, d, best)
        k = jnp.where(c, float(i), k)
        sk = jnp.where(c, si, sk)
        fk = jnp.where(c, fs[i], fk)
    cond = (sk > 0.0) | ((sk == 0.0) & (fk < 0.0))
    nfk = fk + jnp.where(cond, 1.0, -1.0)
    return k, nfk


def _cpe8(xs):
    fs = [_cround(x) for x in xs]
    s0 = fs[0]
    for i in range(1, 8):
        s0 = s0 + fs[i]
    even0 = _is_even(s0)
    k0, nf0 = _g_x_parts(xs, fs)
    k0 = jnp.where(even0, -1.0, k0)
    y0 = [jnp.where(k0 == float(i), nf0, f) for i, f in enumerate(fs)]

    xs2 = [x - 0.5 for x in xs]
    fs2 = [_cround(x) for x in xs2]
    s1 = fs2[0]
    for i in range(1, 8):
        s1 = s1 + fs2[i]
    even1 = _is_even(s1)
    k1, nf1 = _g_x_parts(xs2, fs2)
    k1 = jnp.where(even1, -1.0, k1)
    y1 = [jnp.where(k1 == float(i), nf1, f) + 0.5 for i, f in enumerate(fs2)]

    d0 = (xs[0] - y0[0]) * (xs[0] - y0[0])
    d1 = (xs[0] - y1[0]) * (xs[0] - y1[0])
    for i in range(1, 8):
        d0 = d0 + (xs[i] - y0[i]) * (xs[i] - y0[i])
        d1 = d1 + (xs[i] - y1[i]) * (xs[i] - y1[i])
    c = d0 < d1
    return [jnp.where(c, a, b) for a, b in zip(y0, y1)]


def _encode_coords(xl):
    suf = [None] * 7
    suf[6] = xl[6]
    for j in range(5, 0, -1):
        suf[j] = xl[j] + suf[j + 1]
    c = [None] * 8
    c[0] = 0.5 * (xl[0] + suf[1]) - 3.5 * xl[7]
    for j in range(1, 7):
        c[j] = suf[j] - float(7 - j) * xl[7]
    c[7] = 2.0 * xl[7]
    return c


def _decode_Gb(b):
    h = 0.5 * b[7]
    Gb = [None] * 8
    Gb[0] = 2.0 * b[0] - b[1] + h
    for i in range(1, 6):
        Gb[i] = b[i] - b[i + 1] + h
    Gb[6] = b[6] + h
    Gb[7] = h
    return Gb


def _hnlq_body(beta_ref, eps_ref, x_ref, o_ref):
    beta = beta_ref[0]
    xs = [x_ref[i] for i in range(8)]
    t = [xs[i] / beta + eps_ref[i] for i in range(8)]
    xhat = None
    for layer in range(_M):
        xl = _cpe8(t)
        cc = _encode_coords(xl)
        b = [_digit_mod4(v) for v in cc]
        t = [v * 0.25 for v in xl]
        Gb = _decode_Gb(b)
        gq = _cpe8([v * 0.25 for v in Gb])
        xi = [g - _Q * q for g, q in zip(Gb, gq)]
        if layer == 0:
            xhat = xi
        else:
            w = float(_Q ** layer)
            xhat = [h + w * v for h, v in zip(xhat, xi)]
    for i in range(8):
        o_ref[i] = beta * xhat[i]


def _hnlq_transposed(xt, beta, eps, sb):
    s = xt.shape[1]
    grid = s // sb
    return pl.pallas_call(
        _hnlq_body,
        grid=(grid,),
        in_specs=[
            pl.BlockSpec(memory_space=pltpu.SMEM),
            pl.BlockSpec(memory_space=pltpu.SMEM),
            pl.BlockSpec((8, sb, 128), lambda i: (0, i, 0)),
        ],
        out_specs=pl.BlockSpec((8, sb, 128), lambda i: (0, i, 0)),
        out_shape=jax.ShapeDtypeStruct(xt.shape, jnp.float32),
    )(beta, eps, xt)


def kernel(x, beta, alpha, G, G_inv, eps):
    n = x.shape[0]
    sb = 32
    assert n % (sb * 128) == 0
    xt = x.T.reshape(8, n // 128, 128)
    out = _hnlq_transposed(xt, jnp.reshape(beta, (1,)), eps, sb)
    x_hat = out.reshape(8, n).T
    return x + jax.lax.stop_gradient(x_hat - x)


# speedup vs baseline: 12.7034x; 1.0079x over previous
import jax
import jax.numpy as jnp
from jax.experimental import pallas as pl
from jax.experimental.pallas import tpu as pltpu

_Q = 4.0
_M = 6
_TINY = float(jnp.finfo(jnp.float32).eps)


def _cround(x):
    xb = jax.lax.bitcast_convert_type(x, jnp.uint32)
    st = (xb & jnp.uint32(0x80000000)) | jnp.uint32(0x34000000)
    y = x - jax.lax.bitcast_convert_type(st, jnp.float32)
    return jnp.floor(y + 0.5)


def _is_even(s):
    return (s.astype(jnp.int32) & 1) == 0


def _digit_mod4(v):
    return (v.astype(jnp.int32) & 3).astype(jnp.float32)


def _g_x_parts(xs, fs):
    s = xs[0] - fs[0]
    best = jnp.abs(s)
    k = jnp.zeros_like(best)
    sk = s
    fk = fs[0]
    for i in range(1, 8):
        si = xs[i] - fs[i]
        d = jnp.abs(si)
        c = d > best
        best = jnp.where(c, d, best)
        k = jnp.where(c, float(i), k)
        sk = jnp.where(c, si, sk)
        fk = jnp.where(c, fs[i], fk)
    cond = (sk > 0.0) | ((sk == 0.0) & (fk < 0.0))
    nfk = fk + jnp.where(cond, 1.0, -1.0)
    return k, nfk


def _cpe8(xs):
    fs = [_cround(x) for x in xs]
    s0 = fs[0]
    for i in range(1, 8):
        s0 = s0 + fs[i]
    even0 = _is_even(s0)
    k0, nf0 = _g_x_parts(xs, fs)
    k0 = jnp.where(even0, -1.0, k0)
    y0 = [jnp.where(k0 == float(i), nf0, f) for i, f in enumerate(fs)]

    xs2 = [x - 0.5 for x in xs]
    fs2 = [_cround(x) for x in xs2]
    s1 = fs2[0]
    for i in range(1, 8):
        s1 = s1 + fs2[i]
    even1 = _is_even(s1)
    k1, nf1 = _g_x_parts(xs2, fs2)
    k1 = jnp.where(even1, -1.0, k1)
    y1 = [jnp.where(k1 == float(i), nf1, f) + 0.5 for i, f in enumerate(fs2)]

    d0 = (xs[0] - y0[0]) * (xs[0] - y0[0])
    d1 = (xs[0] - y1[0]) * (xs[0] - y1[0])
    for i in range(1, 8):
        d0 = d0 + (xs[i] - y0[i]) * (xs[i] - y0[i])
        d1 = d1 + (xs[i] - y1[i]) * (xs[i] - y1[i])
    c = d0 < d1
    return [jnp.where(c, a, b) for a, b in zip(y0, y1)]


def _encode_coords(xl):
    suf = [None] * 7
    suf[6] = xl[6]
    for j in range(5, 0, -1):
        suf[j] = xl[j] + suf[j + 1]
    c = [None] * 8
    c[0] = 0.5 * (xl[0] + suf[1]) - 3.5 * xl[7]
    for j in range(1, 7):
        c[j] = suf[j] - float(7 - j) * xl[7]
    c[7] = 2.0 * xl[7]
    return c


def _decode_Gb(b):
    h = 0.5 * b[7]
    Gb = [None] * 8
    Gb[0] = 2.0 * b[0] - b[1] + h
    for i in range(1, 6):
        Gb[i] = b[i] - b[i + 1] + h
    Gb[6] = b[6] + h
    Gb[7] = h
    return Gb


def _hnlq_body(beta_ref, eps_ref, x_ref, o_ref):
    beta = beta_ref[0]
    xs = [x_ref[i] for i in range(8)]
    t = [xs[i] / beta + eps_ref[i] for i in range(8)]
    xhat = None
    for layer in range(_M):
        xl = _cpe8(t)
        cc = _encode_coords(xl)
        b = [_digit_mod4(v) for v in cc]
        t = [v * 0.25 for v in xl]
        Gb = _decode_Gb(b)
        gq = _cpe8([v * 0.25 for v in Gb])
        xi = [g - _Q * q for g, q in zip(Gb, gq)]
        if layer == 0:
            xhat = xi
        else:
            w = float(_Q ** layer)
            xhat = [h + w * v for h, v in zip(xhat, xi)]
    for i in range(8):
        o_ref[i] = beta * xhat[i]


def _hnlq_transposed(xt, beta, eps, sb):
    s = xt.shape[1]
    grid = s // sb
    return pl.pallas_call(
        _hnlq_body,
        grid=(grid,),
        in_specs=[
            pl.BlockSpec(memory_space=pltpu.SMEM),
            pl.BlockSpec(memory_space=pltpu.SMEM),
            pl.BlockSpec((8, sb, 128), lambda i: (0, i, 0)),
        ],
        out_specs=pl.BlockSpec((8, sb, 128), lambda i: (0, i, 0)),
        out_shape=jax.ShapeDtypeStruct(xt.shape, jnp.float32),
    )(beta, eps, xt)


def kernel(x, beta, alpha, G, G_inv, eps):
    n = x.shape[0]
    sb = 64
    assert n % (sb * 128) == 0
    xt = x.T.reshape(8, n // 128, 128)
    out = _hnlq_transposed(xt, jnp.reshape(beta, (1,)), eps, sb)
    x_hat = out.reshape(8, n).T
    return x + jax.lax.stop_gradient(x_hat - x)
